# Initial kernel scaffold; baseline (speedup 1.0000x reference)
import functools

import jax
import jax.numpy as jnp
from jax import lax
from jax.experimental import pallas as pl
from jax.experimental.pallas import tpu as pltpu
from jax.experimental.pallas import tpu_sc as plsc

_HID = 768
_MAX_2D = 1024
_PAD = 1
_EPS = 1e-5
_B, _S = 32, 512
_T = 32
_NCHUNK = _S // _T
_L = 16
_NSL = _HID // _L
_SEG = 128 // _L


def _sc_body(ids_hbm, bbT_hbm, word_hbm, pos_hbm, x_hbm, y_hbm, h_hbm, w_hbm,
             gamma_hbm, beta_hbm, out_hbm,
             ids_v, bb_v, pos_idx_v, h_idx_v, w_idx_v, gamma_v, beta_v,
             word_buf, pos_buf, sx0, sy1, sx2, sy3, sh, sw, sem):
    wid = lax.axis_index("s") * 2 + lax.axis_index("c")
    b = wid

    pltpu.sync_copy(ids_hbm.at[b], ids_v)
    for c in range(4):
        pltpu.sync_copy(bbT_hbm.at[c, b], bb_v.at[c])
    pltpu.sync_copy(gamma_hbm, gamma_v)
    pltpu.sync_copy(beta_hbm, beta_v)

    def idx_body(i, carry):
        sl = pl.ds(i * _L, _L)
        ids = ids_v[sl]
        m = jnp.where(ids == _PAD, 0, 1).astype(jnp.int32)
        cs = lax.cumsum(m, axis=0) + carry
        pos_idx_v[sl] = cs * m + _PAD
        b0 = bb_v[0, sl]
        b1 = bb_v[1, sl]
        b2 = bb_v[2, sl]
        b3 = bb_v[3, sl]
        h_idx_v[sl] = jnp.clip(b3 - b1, 0, _MAX_2D - 1)
        w_idx_v[sl] = jnp.clip(b2 - b0, 0, _MAX_2D - 1)
        return carry + jnp.sum(m)

    lax.fori_loop(0, _S // _L, idx_body, jnp.int32(0))

    spat = (sx0, sy1, sx2, sy3, sh, sw)

    def chunk_body(k, _):
        c0 = k * _T
        sl = pl.ds(c0, _T)
        copies = (
            pltpu.async_copy(word_hbm.at[ids_v.at[sl]], word_buf, sem),
            pltpu.async_copy(pos_hbm.at[pos_idx_v.at[sl]], pos_buf, sem),
            pltpu.async_copy(x_hbm.at[bb_v.at[0, sl]], sx0, sem),
            pltpu.async_copy(y_hbm.at[bb_v.at[1, sl]], sy1, sem),
            pltpu.async_copy(x_hbm.at[bb_v.at[2, sl]], sx2, sem),
            pltpu.async_copy(y_hbm.at[bb_v.at[3, sl]], sy3, sem),
            pltpu.async_copy(h_hbm.at[h_idx_v.at[sl]], sh, sem),
            pltpu.async_copy(w_hbm.at[w_idx_v.at[sl]], sw, sem),
        )
        for cp in copies:
            cp.wait()

        def token_body(t, _):
            vsum = jnp.zeros((_L,), jnp.float32)
            vsq = jnp.zeros((_L,), jnp.float32)
            for s in range(_NSL):
                dsl = pl.ds(s * _L, _L)
                v = word_buf[t, dsl] + pos_buf[t, dsl]
                v = v + spat[s // _SEG][t, pl.ds((s % _SEG) * _L, _L)]
                word_buf[t, dsl] = v
                vsum = vsum + v
                vsq = vsq + v * v
            mean = jnp.sum(vsum) * (1.0 / _HID)
            var = jnp.sum(vsq) * (1.0 / _HID) - mean * mean + _EPS
            xv = jnp.full((_L,), var, dtype=jnp.float32)
            iv = lax.bitcast_convert_type(
                jnp.int32(0x5F3759DF)
                - (lax.bitcast_convert_type(xv, jnp.int32) >> 1),
                jnp.float32)
            for _ in range(3):
                iv = iv * (1.5 - 0.5 * xv * iv * iv)
            for s in range(_NSL):
                dsl = pl.ds(s * _L, _L)
                v = word_buf[t, dsl]
                word_buf[t, dsl] = (v - mean) * iv * gamma_v[dsl] + beta_v[dsl]
            return 0

        lax.fori_loop(0, _T, token_body, 0)
        pltpu.sync_copy(word_buf, out_hbm.at[b, sl])
        return 0

    lax.fori_loop(0, _NCHUNK, chunk_body, 0)


@jax.jit
def _run(input_ids, bbT, word_emb, pos_plus, x_emb, y_emb, h_emb, w_emb,
         ln_gamma, ln_beta):
    k = functools.partial(
        pl.kernel,
        out_type=jax.ShapeDtypeStruct((_B, _S, _HID), jnp.float32),
        mesh=plsc.VectorSubcoreMesh(core_axis_name="c", subcore_axis_name="s"),
        scratch_types=[
            pltpu.VMEM((_S,), jnp.int32),
            pltpu.VMEM((4, _S), jnp.int32),
            pltpu.VMEM((_S,), jnp.int32),
            pltpu.VMEM((_S,), jnp.int32),
            pltpu.VMEM((_S,), jnp.int32),
            pltpu.VMEM((_HID,), jnp.float32),
            pltpu.VMEM((_HID,), jnp.float32),
            pltpu.VMEM((_T, _HID), jnp.float32),
            pltpu.VMEM((_T, _HID), jnp.float32),
            pltpu.VMEM((_T, 128), jnp.float32),
            pltpu.VMEM((_T, 128), jnp.float32),
            pltpu.VMEM((_T, 128), jnp.float32),
            pltpu.VMEM((_T, 128), jnp.float32),
            pltpu.VMEM((_T, 128), jnp.float32),
            pltpu.VMEM((_T, 128), jnp.float32),
            pltpu.SemaphoreType.DMA,
        ],
    )(_sc_body)
    return k(input_ids, bbT, word_emb, pos_plus, x_emb, y_emb, h_emb, w_emb,
             ln_gamma, ln_beta)


def kernel(input_ids, bbox, word_emb, token_type_emb, pos_emb, x_emb, y_emb,
           h_emb, w_emb, ln_gamma, ln_beta):
    pos_plus = pos_emb + token_type_emb[0][None, :]
    bbT = jnp.transpose(bbox, (2, 0, 1))
    return _run(input_ids, bbT, word_emb, pos_plus, x_emb, y_emb, h_emb,
                w_emb, ln_gamma, ln_beta)


# speedup vs baseline: 1.3078x; 1.3078x over previous
import functools

import jax
import jax.numpy as jnp
from jax import lax
from jax.experimental import pallas as pl
from jax.experimental.pallas import tpu as pltpu
from jax.experimental.pallas import tpu_sc as plsc

_HID = 768
_MAX_2D = 1024
_PAD = 1
_EPS = 1e-5
_B, _S = 32, 512
_T = 32
_NCHUNK = _S // _T
_L = 16
_NSL = _HID // _L
_SEG = 128 // _L


def _sc_body(ids_hbm, bbT_hbm, word_hbm, pos_hbm, x_hbm, y_hbm, h_hbm, w_hbm,
             gamma_hbm, beta_hbm, out_hbm,
             ids_v, bb_v, pos_idx_v, h_idx_v, w_idx_v, gamma_v, beta_v,
             word_buf, pos_buf, sx0, sy1, sx2, sy3, sh, sw, sem):
    wid = lax.axis_index("s") * 2 + lax.axis_index("c")
    b = wid

    pltpu.sync_copy(ids_hbm.at[b], ids_v)
    for c in range(4):
        pltpu.sync_copy(bbT_hbm.at[c, b], bb_v.at[c])
    pltpu.sync_copy(gamma_hbm, gamma_v)
    pltpu.sync_copy(beta_hbm, beta_v)

    def idx_body(i, carry):
        sl = pl.ds(i * _L, _L)
        ids = ids_v[sl]
        m = jnp.where(ids == _PAD, 0, 1).astype(jnp.int32)
        cs = lax.cumsum(m, axis=0) + carry
        pos_idx_v[sl] = cs * m + _PAD
        b0 = bb_v[0, sl]
        b1 = bb_v[1, sl]
        b2 = bb_v[2, sl]
        b3 = bb_v[3, sl]
        h_idx_v[sl] = jnp.clip(b3 - b1, 0, _MAX_2D - 1)
        w_idx_v[sl] = jnp.clip(b2 - b0, 0, _MAX_2D - 1)
        return carry + jnp.sum(m)

    lax.fori_loop(0, _S // _L, idx_body, jnp.int32(0))

    spat = (sx0, sy1, sx2, sy3, sh, sw)

    def chunk_body(k, _):
        c0 = k * _T
        sl = pl.ds(c0, _T)
        copies = (
            pltpu.async_copy(word_hbm.at[ids_v.at[sl]], word_buf, sem),
            pltpu.async_copy(pos_hbm.at[pos_idx_v.at[sl]], pos_buf, sem),
            pltpu.async_copy(x_hbm.at[bb_v.at[0, sl]], sx0, sem),
            pltpu.async_copy(y_hbm.at[bb_v.at[1, sl]], sy1, sem),
            pltpu.async_copy(x_hbm.at[bb_v.at[2, sl]], sx2, sem),
            pltpu.async_copy(y_hbm.at[bb_v.at[3, sl]], sy3, sem),
            pltpu.async_copy(h_hbm.at[h_idx_v.at[sl]], sh, sem),
            pltpu.async_copy(w_hbm.at[w_idx_v.at[sl]], sw, sem),
        )
        for cp in copies:
            cp.wait()

        def token_body(t, _):
            vsum = jnp.zeros((_L,), jnp.float32)
            vsq = jnp.zeros((_L,), jnp.float32)
            for s in range(_NSL):
                dsl = pl.ds(s * _L, _L)
                v = word_buf[t, dsl] + pos_buf[t, dsl]
                v = v + spat[s // _SEG][t, pl.ds((s % _SEG) * _L, _L)]
                word_buf[t, dsl] = v
                vsum = vsum + v
                vsq = vsq + v * v
            mean = jnp.sum(vsum) * (1.0 / _HID)
            var = jnp.sum(vsq) * (1.0 / _HID) - mean * mean + _EPS
            xv = jnp.full((_L,), var, dtype=jnp.float32)
            iv = lax.bitcast_convert_type(
                jnp.int32(0x5F3759DF)
                - (lax.bitcast_convert_type(xv, jnp.int32) >> 1),
                jnp.float32)
            for _ in range(3):
                iv = iv * (1.5 - 0.5 * xv * iv * iv)
            for s in range(_NSL):
                dsl = pl.ds(s * _L, _L)
                v = word_buf[t, dsl]
                word_buf[t, dsl] = (v - mean) * iv * gamma_v[dsl] + beta_v[dsl]
            return 0

        lax.fori_loop(0, _T, token_body, 0)
        pltpu.sync_copy(word_buf, out_hbm.at[b, sl])
        return 0

    lax.fori_loop(0, _NCHUNK, chunk_body, 0)


@jax.jit
def _run(input_ids, bbT, word_emb, pos_plus, x_emb, y_emb, h_emb, w_emb,
         ln_gamma, ln_beta):
    k = functools.partial(
        pl.kernel,
        out_type=jax.ShapeDtypeStruct((_B, _S, _HID), jnp.float32),
        mesh=plsc.VectorSubcoreMesh(core_axis_name="c", subcore_axis_name="s"),
        compiler_params=pltpu.CompilerParams(needs_layout_passes=False),
        scratch_types=[
            pltpu.VMEM((_S,), jnp.int32),
            pltpu.VMEM((4, _S), jnp.int32),
            pltpu.VMEM((_S,), jnp.int32),
            pltpu.VMEM((_S,), jnp.int32),
            pltpu.VMEM((_S,), jnp.int32),
            pltpu.VMEM((_HID,), jnp.float32),
            pltpu.VMEM((_HID,), jnp.float32),
            pltpu.VMEM((_T, _HID), jnp.float32),
            pltpu.VMEM((_T, _HID), jnp.float32),
            pltpu.VMEM((_T, 128), jnp.float32),
            pltpu.VMEM((_T, 128), jnp.float32),
            pltpu.VMEM((_T, 128), jnp.float32),
            pltpu.VMEM((_T, 128), jnp.float32),
            pltpu.VMEM((_T, 128), jnp.float32),
            pltpu.VMEM((_T, 128), jnp.float32),
            pltpu.SemaphoreType.DMA,
        ],
    )(_sc_body)
    return k(input_ids, bbT, word_emb, pos_plus, x_emb, y_emb, h_emb, w_emb,
             ln_gamma, ln_beta)


def kernel(input_ids, bbox, word_emb, token_type_emb, pos_emb, x_emb, y_emb,
           h_emb, w_emb, ln_gamma, ln_beta):
    pos_plus = pos_emb + token_type_emb[0][None, :]
    bbT = jnp.transpose(bbox, (2, 0, 1))
    return _run(input_ids, bbT, word_emb, pos_plus, x_emb, y_emb, h_emb,
                w_emb, ln_gamma, ln_beta)


# speedup vs baseline: 1.3555x; 1.0365x over previous
import functools

import jax
import jax.numpy as jnp
from jax import lax
from jax.experimental import pallas as pl
from jax.experimental.pallas import tpu as pltpu
from jax.experimental.pallas import tpu_sc as plsc

_HID = 768
_MAX_2D = 1024
_PAD = 1
_EPS = 1e-5
_B, _S = 32, 512
_T = 16
_NCHUNK = _S // _T
_L = 16
_NSL = _HID // _L
_SEG = 128 // _L


def _sc_body(ids_hbm, bbT_hbm, word_hbm, pos_hbm, x_hbm, y_hbm, h_hbm, w_hbm,
             gamma_hbm, beta_hbm, out_hbm,
             ids_v, bb_v, pos_idx_v, h_idx_v, w_idx_v, gamma_v, beta_v,
             bufs_a, bufs_b, sem_a, sem_b, out_sem_a, out_sem_b):
    wid = lax.axis_index("s") * 2 + lax.axis_index("c")
    b = wid

    pltpu.sync_copy(ids_hbm.at[b], ids_v)
    for c in range(4):
        pltpu.sync_copy(bbT_hbm.at[c, b], bb_v.at[c])
    pltpu.sync_copy(gamma_hbm, gamma_v)
    pltpu.sync_copy(beta_hbm, beta_v)

    def idx_body(i, carry):
        sl = pl.ds(i * _L, _L)
        ids = ids_v[sl]
        m = jnp.where(ids == _PAD, 0, 1).astype(jnp.int32)
        cs = lax.cumsum(m, axis=0) + carry
        pos_idx_v[sl] = cs * m + _PAD
        b0 = bb_v[0, sl]
        b1 = bb_v[1, sl]
        b2 = bb_v[2, sl]
        b3 = bb_v[3, sl]
        h_idx_v[sl] = jnp.clip(b3 - b1, 0, _MAX_2D - 1)
        w_idx_v[sl] = jnp.clip(b2 - b0, 0, _MAX_2D - 1)
        return carry + jnp.sum(m)

    lax.fori_loop(0, _S // _L, idx_body, jnp.int32(0))

    def issue(c0, bufs, sem):
        sl = pl.ds(c0, _T)
        word_buf, pos_buf, sx0, sy1, sx2, sy3, sh, sw = bufs
        pltpu.async_copy(word_hbm.at[ids_v.at[sl]], word_buf, sem)
        pltpu.async_copy(pos_hbm.at[pos_idx_v.at[sl]], pos_buf, sem)
        pltpu.async_copy(x_hbm.at[bb_v.at[0, sl]], sx0, sem)
        pltpu.async_copy(y_hbm.at[bb_v.at[1, sl]], sy1, sem)
        pltpu.async_copy(x_hbm.at[bb_v.at[2, sl]], sx2, sem)
        pltpu.async_copy(y_hbm.at[bb_v.at[3, sl]], sy3, sem)
        pltpu.async_copy(h_hbm.at[h_idx_v.at[sl]], sh, sem)
        pltpu.async_copy(w_hbm.at[w_idx_v.at[sl]], sw, sem)

    def wait_gathers(bufs, sem):
        word_buf, pos_buf, sx0, sy1, sx2, sy3, sh, sw = bufs
        dummy = pl.ds(0, _T)
        pltpu.make_async_copy(word_hbm.at[dummy], word_buf, sem).wait()
        pltpu.make_async_copy(pos_hbm.at[dummy], pos_buf, sem).wait()
        pltpu.make_async_copy(x_hbm.at[dummy], sx0, sem).wait()
        pltpu.make_async_copy(y_hbm.at[dummy], sy1, sem).wait()
        pltpu.make_async_copy(x_hbm.at[dummy], sx2, sem).wait()
        pltpu.make_async_copy(y_hbm.at[dummy], sy3, sem).wait()
        pltpu.make_async_copy(h_hbm.at[dummy], sh, sem).wait()
        pltpu.make_async_copy(w_hbm.at[dummy], sw, sem).wait()

    def compute(bufs):
        word_buf, pos_buf, sx0, sy1, sx2, sy3, sh, sw = bufs
        spat = (sx0, sy1, sx2, sy3, sh, sw)

        def token_body(t, _):
            vsum = jnp.zeros((_L,), jnp.float32)
            vsq = jnp.zeros((_L,), jnp.float32)
            for s in range(_NSL):
                dsl = pl.ds(s * _L, _L)
                v = word_buf[t, dsl] + pos_buf[t, dsl]
                v = v + spat[s // _SEG][t, pl.ds((s % _SEG) * _L, _L)]
                word_buf[t, dsl] = v
                vsum = vsum + v
                vsq = vsq + v * v
            mean = jnp.sum(vsum) * (1.0 / _HID)
            var = jnp.sum(vsq) * (1.0 / _HID) - mean * mean + _EPS
            xv = jnp.full((_L,), var, dtype=jnp.float32)
            iv = lax.bitcast_convert_type(
                jnp.int32(0x5F3759DF)
                - (lax.bitcast_convert_type(xv, jnp.int32) >> 1),
                jnp.float32)
            for _ in range(3):
                iv = iv * (1.5 - 0.5 * xv * iv * iv)
            for s in range(_NSL):
                dsl = pl.ds(s * _L, _L)
                v = word_buf[t, dsl]
                word_buf[t, dsl] = (v - mean) * iv * gamma_v[dsl] + beta_v[dsl]
            return 0

        lax.fori_loop(0, _T, token_body, 0)

    def out_copy(c0, bufs, osem):
        pltpu.async_copy(bufs[0], out_hbm.at[b, pl.ds(c0, _T)], osem)

    def wait_out(bufs, osem):
        pltpu.make_async_copy(bufs[0], out_hbm.at[b, pl.ds(0, _T)], osem).wait()

    issue(0, bufs_a, sem_a)
    issue(_T, bufs_b, sem_b)
    wait_gathers(bufs_a, sem_a)
    compute(bufs_a)
    out_copy(0, bufs_a, out_sem_a)

    def pipe_body(j, _):
        c1 = (1 + 2 * j) * _T
        wait_out(bufs_a, out_sem_a)
        issue(c1 + _T, bufs_a, sem_a)
        wait_gathers(bufs_b, sem_b)
        compute(bufs_b)
        out_copy(c1, bufs_b, out_sem_b)

        c2 = c1 + _T
        wait_out(bufs_b, out_sem_b)
        issue(c2 + _T, bufs_b, sem_b)
        wait_gathers(bufs_a, sem_a)
        compute(bufs_a)
        out_copy(c2, bufs_a, out_sem_a)
        return 0

    lax.fori_loop(0, (_NCHUNK - 2) // 2, pipe_body, 0)

    wait_gathers(bufs_b, sem_b)
    compute(bufs_b)
    out_copy((_NCHUNK - 1) * _T, bufs_b, out_sem_b)
    wait_out(bufs_a, out_sem_a)
    wait_out(bufs_b, out_sem_b)


def _buf_set():
    return (
        pltpu.VMEM((_T, _HID), jnp.float32),
        pltpu.VMEM((_T, _HID), jnp.float32),
        pltpu.VMEM((_T, 128), jnp.float32),
        pltpu.VMEM((_T, 128), jnp.float32),
        pltpu.VMEM((_T, 128), jnp.float32),
        pltpu.VMEM((_T, 128), jnp.float32),
        pltpu.VMEM((_T, 128), jnp.float32),
        pltpu.VMEM((_T, 128), jnp.float32),
    )


@jax.jit
def _run(input_ids, bbT, word_emb, pos_plus, x_emb, y_emb, h_emb, w_emb,
         ln_gamma, ln_beta):
    k = functools.partial(
        pl.kernel,
        out_type=jax.ShapeDtypeStruct((_B, _S, _HID), jnp.float32),
        mesh=plsc.VectorSubcoreMesh(core_axis_name="c", subcore_axis_name="s"),
        compiler_params=pltpu.CompilerParams(needs_layout_passes=False),
        scratch_types=[
            pltpu.VMEM((_S,), jnp.int32),
            pltpu.VMEM((4, _S), jnp.int32),
            pltpu.VMEM((_S,), jnp.int32),
            pltpu.VMEM((_S,), jnp.int32),
            pltpu.VMEM((_S,), jnp.int32),
            pltpu.VMEM((_HID,), jnp.float32),
            pltpu.VMEM((_HID,), jnp.float32),
            _buf_set(),
            _buf_set(),
            pltpu.SemaphoreType.DMA,
            pltpu.SemaphoreType.DMA,
            pltpu.SemaphoreType.DMA,
            pltpu.SemaphoreType.DMA,
        ],
    )(_sc_body)
    return k(input_ids, bbT, word_emb, pos_plus, x_emb, y_emb, h_emb, w_emb,
             ln_gamma, ln_beta)


def kernel(input_ids, bbox, word_emb, token_type_emb, pos_emb, x_emb, y_emb,
           h_emb, w_emb, ln_gamma, ln_beta):
    pos_plus = pos_emb + token_type_emb[0][None, :]
    bbT = jnp.transpose(bbox, (2, 0, 1))
    return _run(input_ids, bbT, word_emb, pos_plus, x_emb, y_emb, h_emb,
                w_emb, ln_gamma, ln_beta)
